# Initial kernel scaffold; baseline (speedup 1.0000x reference)
import jax
import jax.numpy as jnp
from jax.experimental import pallas as pl


def kernel(x, attn, as_out, cluster_num):
    raise NotImplementedError("write your pallas kernel here")


# speedup vs baseline: 2.3301x; 2.3301x over previous
import jax
import jax.numpy as jnp
from jax.experimental import pallas as pl
from jax.experimental.pallas import tpu as pltpu

_N = 1024
_C = 192
_H = 8
_CN = 256
_K = 5
_ALPHA = 0.2
_SQRT_C = float(_C ** 0.5)


def _atm_body(x_ref, attn_ref, extras_ref, xm_ref, idx_ref, acc_ref):
    h = pl.program_id(1)

    @pl.when(h == 0)
    def _init():
        acc_ref[...] = attn_ref[0, 0]

    @pl.when(h > 0)
    def _accum():
        acc_ref[...] = acc_ref[...] + attn_ref[0, 0]

    @pl.when(h == _H - 1)
    def _compute():
        X = x_ref[0]
        A = acc_ref[...]

        n1 = jnp.sum(X * X, axis=1)
        g1 = jax.lax.dot_general(X, X, (((1,), (1,)), ((), ())),
                                 preferred_element_type=jnp.float32)
        d1 = jnp.sqrt(jnp.maximum(n1[:, None] + n1[None, :] - 2.0 * g1, 0.0))
        n2 = jnp.sum(A * A, axis=1)
        g2 = jax.lax.dot_general(A, A, (((1,), (1,)), ((), ())),
                                 preferred_element_type=jnp.float32)
        d2 = jnp.sqrt(jnp.maximum(n2[:, None] + n2[None, :] - 2.0 * g2, 0.0))
        dist = (1.0 - _ALPHA) * (d1 / _SQRT_C) + _ALPHA * (d2 / _SQRT_C)
        dist_max = jnp.max(dist, keepdims=True).reshape(1, 1)

        s = jnp.zeros((1, _N), jnp.float32)
        rem = jnp.full((1, _N), float(_K), jnp.float32)
        m = jnp.full((1, _N), -jnp.inf, jnp.float32)
        for _ in range(_K):
            cand = jnp.where(dist > m, dist, jnp.inf)
            m = jnp.min(cand, axis=0, keepdims=True)
            c = jnp.sum((dist == m).astype(jnp.float32), axis=0, keepdims=True)
            t = jnp.minimum(c, rem)
            s = s + jnp.where(t > 0.0, m * m * t, 0.0)
            rem = rem - t
        noise = extras_ref[0, 0:1, :]
        w_as = extras_ref[0, 1:2, :]
        dens = jnp.exp(-(s / float(_K))) + noise
        dens_col = dens.reshape(_N, 1)

        masked = jnp.where(dens_col > dens, dist, dist_max)
        dist_min = jnp.min(masked, axis=0, keepdims=True)

        score = dist_min * dens + w_as
        iota_n = jax.lax.broadcasted_iota(jnp.int32, (1, _N), 1)
        iota_cn = jax.lax.broadcasted_iota(jnp.int32, (1, _CN), 1)

        def topk_step(j, carry):
            sc, idxv = carry
            mx = jnp.max(sc, axis=1, keepdims=True)
            amx = jnp.min(jnp.where(sc == mx, iota_n, _N), axis=1,
                          keepdims=True)
            idxv = jnp.where(iota_cn == j, amx, idxv)
            sc = jnp.where(iota_n == amx, -jnp.inf, sc)
            return sc, idxv

        _, idx_down = jax.lax.fori_loop(
            0, _CN, topk_step, (score, jnp.zeros((1, _CN), jnp.int32)))

        iota_rows = jax.lax.broadcasted_iota(jnp.int32, (_N, _CN), 0)
        iota_cols = jax.lax.broadcasted_iota(jnp.int32, (_N, _CN), 1)
        onehot = (iota_rows == idx_down).astype(jnp.float32)
        dmc = jax.lax.dot_general(dist, onehot, (((1,), (0,)), ((), ())),
                                  preferred_element_type=jnp.float32,
                                  precision=jax.lax.Precision.HIGHEST)
        mn = jnp.min(dmc, axis=1, keepdims=True)
        amn = jnp.min(jnp.where(dmc == mn, iota_cols, _CN), axis=1,
                      keepdims=True)
        is_center = jnp.sum(onehot, axis=1, keepdims=True) > 0.0
        jpos = jnp.sum(onehot * iota_cols.astype(jnp.float32), axis=1,
                       keepdims=True)
        idx_cluster = jnp.where(is_center, jpos.astype(jnp.int32), amn)

        assign = (iota_cols == idx_cluster).astype(jnp.float32)
        counts = jnp.sum(assign, axis=0, keepdims=True)
        sums = jax.lax.dot_general(assign, X, (((0,), (0,)), ((), ())),
                                   preferred_element_type=jnp.float32,
                                   precision=jax.lax.Precision.HIGHEST)
        xm_ref[0] = sums / (counts.reshape(_CN, 1) + 1e-06)
        idx_ref[0] = idx_cluster.reshape(1, _N)


def kernel(x, attn, as_out, cluster_num):
    B, N, C = x.shape
    weight = as_out.reshape(B, -1).astype(x.dtype)
    noise = jax.random.uniform(jax.random.key(1), (B, N), dtype=x.dtype) * 1e-06
    extras = jnp.stack([noise, weight], axis=1)
    xm, idx = pl.pallas_call(
        _atm_body,
        grid=(B, _H),
        in_specs=[
            pl.BlockSpec((1, N, C), lambda b, h: (b, 0, 0)),
            pl.BlockSpec((1, 1, N, N), lambda b, h: (b, h, 0, 0)),
            pl.BlockSpec((1, 2, N), lambda b, h: (b, 0, 0)),
        ],
        out_specs=[
            pl.BlockSpec((1, _CN, C), lambda b, h: (b, 0, 0)),
            pl.BlockSpec((1, 1, N), lambda b, h: (b, 0, 0)),
        ],
        out_shape=[
            jax.ShapeDtypeStruct((B, _CN, C), x.dtype),
            jax.ShapeDtypeStruct((B, 1, N), jnp.int32),
        ],
        scratch_shapes=[pltpu.VMEM((_N, _N), jnp.float32)],
        compiler_params=pltpu.CompilerParams(
            dimension_semantics=("arbitrary", "arbitrary")),
    )(x, attn, extras)
    return xm, idx.reshape(B, N)
